# baseline (speedup 1.0000x reference)
import functools

import jax
import jax.numpy as jnp
from jax import lax
from jax.experimental import pallas as pl
from jax.experimental.pallas import tpu as pltpu
from jax.experimental.pallas import tpu_sc as plsc

_B = 16
_L = 2048
_D = 256
_SEQ = 2 * _L + 1
_ROWS = _B * _SEQ
_NW = 32
_HALF = _L // 2


def _sc_body(item_hbm, action_hbm, ctx_hbm, out_hbm, sem):
    c = lax.axis_index("c")
    s = lax.axis_index("s")
    w = s * 2 + c
    b = w // 2
    h = w % 2
    s0 = b * _L + h * _HALF
    dbase = b * _SEQ + 1 + h * _L
    q0 = dbase // 2
    odd = dbase % 2

    cp_i = pltpu.make_async_copy(
        item_hbm.at[pl.ds(s0, _HALF)],
        out_hbm.at[pl.ds(q0, _HALF), pl.ds(odd, 1), :],
        sem,
    )
    cp_i.start()
    cp_a = pltpu.make_async_copy(
        action_hbm.at[pl.ds(s0, _HALF)],
        out_hbm.at[pl.ds(q0 + odd, _HALF), pl.ds(1 - odd, 1), :],
        sem,
    )
    cp_a.start()

    @pl.when(h == 0)
    def _():
        r = b * _SEQ
        pltpu.sync_copy(
            ctx_hbm.at[pl.ds(b, 1)],
            out_hbm.at[pl.ds(r // 2, 1), pl.ds(r % 2, 1), :],
        )

    cp_i.wait()
    cp_a.wait()


def kernel(item_values, action_values, contextual_values):
    mesh = plsc.VectorSubcoreMesh(core_axis_name="c", subcore_axis_name="s")
    run = functools.partial(
        pl.kernel,
        out_type=jax.ShapeDtypeStruct((_ROWS // 2, 2, _D), jnp.float32),
        mesh=mesh,
        scratch_types=[pltpu.SemaphoreType.DMA],
    )(_sc_body)
    out3 = run(
        item_values.reshape(_B * _L, 1, _D),
        action_values.reshape(_B * _L, 1, _D),
        contextual_values.reshape(_B, 1, _D),
    )
    out_values = out3.reshape(_ROWS, _D)
    out_lengths = jnp.full((_B,), _SEQ, dtype=jnp.int32)
    out_offsets = jnp.concatenate(
        [jnp.zeros((1,), jnp.int32), jnp.cumsum(out_lengths).astype(jnp.int32)]
    )
    return out_values, out_lengths, out_offsets


# speedup vs baseline: 10.7411x; 10.7411x over previous
import functools

import jax
import jax.numpy as jnp
from jax import lax
from jax.experimental import pallas as pl
from jax.experimental.pallas import tpu as pltpu
from jax.experimental.pallas import tpu_sc as plsc

_B = 16
_L = 2048
_D = 256
_SEQ = 2 * _L + 1
_ROWS = _B * _SEQ
_HALF = _L // 2
_C = 32
_K = _HALF // _C
_NBUF = 4
_PIPE = 2


def _pipelined_chunks(chunk_specs, bufs, gsems, ssems):
    K = len(chunk_specs)
    gathers = {}
    scatters = {}

    def start_gathers(k):
        gathers[k] = [g(k % _NBUF) for g in chunk_specs[k][0]]

    def start_scatter(k):
        scatters[k] = chunk_specs[k][1](k % _NBUF)

    for k in range(min(_PIPE, K)):
        start_gathers(k)
    waited = set()
    for k in range(K):
        for cp in gathers[k]:
            cp.wait()
        start_scatter(k)
        nk = k + _PIPE
        if nk < K:
            if nk >= _NBUF:
                scatters[nk - _NBUF].wait()
                waited.add(nk - _NBUF)
            start_gathers(nk)
    for k in range(K):
        if k not in waited:
            scatters[k].wait()


def _sc_body(item_hbm, action_hbm, ctx_hbm, out_hbm, bufs, gsems, ssems):
    c = lax.axis_index("c")
    s = lax.axis_index("s")
    w = s * 2 + c
    b = w // 2
    h = w % 2
    s0 = b * _L + h * _HALF
    dbase = b * _SEQ + 1 + h * _L
    q0 = dbase // 2
    p = dbase % 2

    def gather(src, src_row, slot, col, n):
        cp = pltpu.make_async_copy(
            src.at[pl.ds(src_row, n)],
            bufs[slot].at[pl.ds(0, n), pl.ds(col, 1), :],
            gsems[slot],
        )
        cp.start()
        return cp

    def scatter(slot, dst_row, n):
        cp = pltpu.make_async_copy(
            bufs[slot].at[pl.ds(0, n)],
            out_hbm.at[pl.ds(dst_row, n)],
            ssems[slot],
        )
        cp.start()
        return cp

    @pl.when(h == 0)
    def _():
        r = b * _SEQ
        pltpu.sync_copy(
            ctx_hbm.at[pl.ds(b, 1)],
            out_hbm.at[pl.ds(r // 2, 1), pl.ds(r % 2, 1), :],
        )

    @pl.when(p == 0)
    def _():
        specs = []
        for k in range(_K):
            m0 = k * _C
            specs.append((
                [
                    functools.partial(gather, item_hbm, s0 + m0, col=0, n=_C),
                    functools.partial(gather, action_hbm, s0 + m0, col=1, n=_C),
                ],
                functools.partial(scatter, dst_row=q0 + m0, n=_C),
            ))
        _pipelined_chunks(specs, bufs, gsems, ssems)

    @pl.when(p == 1)
    def _():
        specs = []
        for k in range(_K):
            m0 = 1 + k * _C
            n = _C if k < _K - 1 else _C - 1
            specs.append((
                [
                    functools.partial(gather, action_hbm, s0 + m0 - 1, col=0, n=n),
                    functools.partial(gather, item_hbm, s0 + m0, col=1, n=n),
                ],
                functools.partial(scatter, dst_row=q0 + m0, n=n),
            ))
        _pipelined_chunks(specs, bufs, gsems, ssems)
        pltpu.sync_copy(
            item_hbm.at[pl.ds(s0, 1)],
            out_hbm.at[pl.ds(q0, 1), pl.ds(1, 1), :],
        )
        pltpu.sync_copy(
            action_hbm.at[pl.ds(s0 + _HALF - 1, 1)],
            out_hbm.at[pl.ds(q0 + _HALF, 1), pl.ds(0, 1), :],
        )


def kernel(item_values, action_values, contextual_values):
    mesh = plsc.VectorSubcoreMesh(core_axis_name="c", subcore_axis_name="s")
    body = lambda *refs: _sc_body(
        refs[0], refs[1], refs[2], refs[3],
        list(refs[4:4 + _NBUF]),
        list(refs[4 + _NBUF:4 + 2 * _NBUF]),
        list(refs[4 + 2 * _NBUF:4 + 3 * _NBUF]),
    )
    run = functools.partial(
        pl.kernel,
        out_type=jax.ShapeDtypeStruct((_ROWS // 2, 2, _D), jnp.float32),
        mesh=mesh,
        scratch_types=(
            [pltpu.VMEM((_C, 2, _D), jnp.float32)] * _NBUF
            + [pltpu.SemaphoreType.DMA] * (2 * _NBUF)
        ),
    )(body)
    out3 = run(
        item_values.reshape(_B * _L, 1, _D),
        action_values.reshape(_B * _L, 1, _D),
        contextual_values.reshape(_B, 1, _D),
    )
    out_values = out3.reshape(_ROWS, _D)
    out_lengths = jnp.full((_B,), _SEQ, dtype=jnp.int32)
    out_offsets = jnp.concatenate(
        [jnp.zeros((1,), jnp.int32), jnp.cumsum(out_lengths).astype(jnp.int32)]
    )
    return out_values, out_lengths, out_offsets


# speedup vs baseline: 10.8687x; 1.0119x over previous
import functools

import jax
import jax.numpy as jnp
from jax import lax
from jax.experimental import pallas as pl
from jax.experimental.pallas import tpu as pltpu
from jax.experimental.pallas import tpu_sc as plsc

_B = 16
_L = 2048
_D = 256
_SEQ = 2 * _L + 1
_ROWS = _B * _SEQ
_HALF = _L // 2
_C = 64
_K = _HALF // _C
_NBUF = 3
_PIPE = 2


def _pipelined_chunks(chunk_specs, bufs, gsems, ssems):
    K = len(chunk_specs)
    gathers = {}
    scatters = {}

    def start_gathers(k):
        gathers[k] = [g(k % _NBUF) for g in chunk_specs[k][0]]

    def start_scatter(k):
        scatters[k] = chunk_specs[k][1](k % _NBUF)

    for k in range(min(_PIPE, K)):
        start_gathers(k)
    waited = set()
    for k in range(K):
        for cp in gathers[k]:
            cp.wait()
        start_scatter(k)
        nk = k + _PIPE
        if nk < K:
            if nk >= _NBUF:
                scatters[nk - _NBUF].wait()
                waited.add(nk - _NBUF)
            start_gathers(nk)
    for k in range(K):
        if k not in waited:
            scatters[k].wait()


def _sc_body(item_hbm, action_hbm, ctx_hbm, out_hbm, bufs, gsems, ssems):
    c = lax.axis_index("c")
    s = lax.axis_index("s")
    w = s * 2 + c
    b = w // 2
    h = w % 2
    s0 = b * _L + h * _HALF
    dbase = b * _SEQ + 1 + h * _L
    q0 = dbase // 2
    p = dbase % 2

    def gather(src, src_row, slot, col, n):
        cp = pltpu.make_async_copy(
            src.at[pl.ds(src_row, n)],
            bufs[slot].at[pl.ds(0, n), pl.ds(col, 1), :],
            gsems[slot],
        )
        cp.start()
        return cp

    def scatter(slot, dst_row, n):
        cp = pltpu.make_async_copy(
            bufs[slot].at[pl.ds(0, n)],
            out_hbm.at[pl.ds(dst_row, n)],
            ssems[slot],
        )
        cp.start()
        return cp

    @pl.when(h == 0)
    def _():
        r = b * _SEQ
        pltpu.sync_copy(
            ctx_hbm.at[pl.ds(b, 1)],
            out_hbm.at[pl.ds(r // 2, 1), pl.ds(r % 2, 1), :],
        )

    @pl.when(p == 0)
    def _():
        specs = []
        for k in range(_K):
            m0 = k * _C
            specs.append((
                [
                    functools.partial(gather, item_hbm, s0 + m0, col=0, n=_C),
                    functools.partial(gather, action_hbm, s0 + m0, col=1, n=_C),
                ],
                functools.partial(scatter, dst_row=q0 + m0, n=_C),
            ))
        _pipelined_chunks(specs, bufs, gsems, ssems)

    @pl.when(p == 1)
    def _():
        specs = []
        for k in range(_K):
            m0 = 1 + k * _C
            n = _C if k < _K - 1 else _C - 1
            specs.append((
                [
                    functools.partial(gather, action_hbm, s0 + m0 - 1, col=0, n=n),
                    functools.partial(gather, item_hbm, s0 + m0, col=1, n=n),
                ],
                functools.partial(scatter, dst_row=q0 + m0, n=n),
            ))
        _pipelined_chunks(specs, bufs, gsems, ssems)
        pltpu.sync_copy(
            item_hbm.at[pl.ds(s0, 1)],
            out_hbm.at[pl.ds(q0, 1), pl.ds(1, 1), :],
        )
        pltpu.sync_copy(
            action_hbm.at[pl.ds(s0 + _HALF - 1, 1)],
            out_hbm.at[pl.ds(q0 + _HALF, 1), pl.ds(0, 1), :],
        )


def kernel(item_values, action_values, contextual_values):
    mesh = plsc.VectorSubcoreMesh(core_axis_name="c", subcore_axis_name="s")
    body = lambda *refs: _sc_body(
        refs[0], refs[1], refs[2], refs[3],
        list(refs[4:4 + _NBUF]),
        list(refs[4 + _NBUF:4 + 2 * _NBUF]),
        list(refs[4 + 2 * _NBUF:4 + 3 * _NBUF]),
    )
    run = functools.partial(
        pl.kernel,
        out_type=jax.ShapeDtypeStruct((_ROWS // 2, 2, _D), jnp.float32),
        mesh=mesh,
        scratch_types=(
            [pltpu.VMEM((_C, 2, _D), jnp.float32)] * _NBUF
            + [pltpu.SemaphoreType.DMA] * (2 * _NBUF)
        ),
    )(body)
    out3 = run(
        item_values.reshape(_B * _L, 1, _D),
        action_values.reshape(_B * _L, 1, _D),
        contextual_values.reshape(_B, 1, _D),
    )
    out_values = out3.reshape(_ROWS, _D)
    out_lengths = jnp.full((_B,), _SEQ, dtype=jnp.int32)
    out_offsets = jnp.concatenate(
        [jnp.zeros((1,), jnp.int32), jnp.cumsum(out_lengths).astype(jnp.int32)]
    )
    return out_values, out_lengths, out_offsets


# speedup vs baseline: 30.8085x; 2.8346x over previous
import functools

import jax
import jax.numpy as jnp
from jax import lax
from jax.experimental import pallas as pl
from jax.experimental.pallas import tpu as pltpu
from jax.experimental.pallas import tpu_sc as plsc

_B = 16
_L = 2048
_D = 256
_SEQ = 2 * _L + 1
_ROWS = _B * _SEQ
_HALF = _L // 2
_C = 64
_K = _HALF // _C
_NBUF = 3
_PIPE = 2


def _pipelined_chunks(chunk_specs):
    K = len(chunk_specs)
    gathers = {}
    scatters = {}

    def start_gathers(k):
        gathers[k] = [g(k % _NBUF) for g in chunk_specs[k][0]]

    def start_scatters(k):
        scatters[k] = [s(k % _NBUF) for s in chunk_specs[k][1]]

    for k in range(min(_PIPE, K)):
        start_gathers(k)
    waited = set()
    for k in range(K):
        for cp in gathers[k]:
            cp.wait()
        start_scatters(k)
        nk = k + _PIPE
        if nk < K:
            if nk >= _NBUF:
                for cp in scatters[nk - _NBUF]:
                    cp.wait()
                waited.add(nk - _NBUF)
            start_gathers(nk)
    for k in range(K):
        if k not in waited:
            for cp in scatters[k]:
                cp.wait()


def _iota16():
    return lax.iota(jnp.int32, 16)


def _sc_body(item_hbm, action_hbm, ctx_hbm, out_hbm, *refs):
    bufs_i = list(refs[0:_NBUF])
    bufs_a = list(refs[_NBUF:2 * _NBUF])
    idx_i = list(refs[2 * _NBUF:3 * _NBUF])
    idx_a = list(refs[3 * _NBUF:4 * _NBUF])
    buf_c = refs[4 * _NBUF]
    idx_c = refs[4 * _NBUF + 1]
    gsems = list(refs[4 * _NBUF + 2:4 * _NBUF + 2 + _NBUF])
    ssems = list(refs[4 * _NBUF + 2 + _NBUF:4 * _NBUF + 2 + 2 * _NBUF])
    csem = refs[4 * _NBUF + 2 + 2 * _NBUF]

    c = lax.axis_index("c")
    s = lax.axis_index("s")
    w = s * 2 + c
    b = w // 2
    h = w % 2
    s0 = b * _L + h * _HALF
    dbase = b * _SEQ + 1 + h * _L

    @pl.when(w == 0)
    def _():
        cp = pltpu.make_async_copy(ctx_hbm, buf_c, csem)
        cp.start()
        idx_c[pl.ds(0, 16)] = _SEQ * _iota16()
        cp.wait()
        cp2 = pltpu.make_async_copy(buf_c, out_hbm.at[idx_c], csem)
        cp2.start()
        cp2.wait()

    def gather(src, base, bufs, slot):
        cp = pltpu.make_async_copy(
            src.at[pl.ds(pl.multiple_of(base, 8), _C)], bufs[slot], gsems[slot])
        cp.start()
        return cp

    def scatter(bufs, idxs, dst0, step_off, slot):
        for v in range(_C // 16):
            idxs[slot][pl.ds(16 * v, 16)] = (
                dst0 + 2 * (16 * v + _iota16()) + step_off)
        cp = pltpu.make_async_copy(bufs[slot], out_hbm.at[idxs[slot]], ssems[slot])
        cp.start()
        return cp

    specs = []
    for k in range(_K):
        src0 = s0 + k * _C
        dst0 = dbase + 2 * k * _C
        specs.append((
            [
                functools.partial(gather, item_hbm, src0, bufs_i),
                functools.partial(gather, action_hbm, src0, bufs_a),
            ],
            [
                functools.partial(scatter, bufs_i, idx_i, dst0, 0),
                functools.partial(scatter, bufs_a, idx_a, dst0, 1),
            ],
        ))
    _pipelined_chunks(specs)


def kernel(item_values, action_values, contextual_values):
    mesh = plsc.VectorSubcoreMesh(core_axis_name="c", subcore_axis_name="s")
    run = functools.partial(
        pl.kernel,
        out_type=jax.ShapeDtypeStruct((_ROWS, _D), jnp.float32),
        mesh=mesh,
        scratch_types=(
            [pltpu.VMEM((_C, _D), jnp.float32)] * (2 * _NBUF)
            + [pltpu.VMEM((_C,), jnp.int32)] * (2 * _NBUF)
            + [pltpu.VMEM((_B, _D), jnp.float32)]
            + [pltpu.VMEM((_B,), jnp.int32)]
            + [pltpu.SemaphoreType.DMA] * (2 * _NBUF + 1)
        ),
    )(_sc_body)
    out_values = run(item_values, action_values, contextual_values)
    out_lengths = jnp.full((_B,), _SEQ, dtype=jnp.int32)
    out_offsets = jnp.concatenate(
        [jnp.zeros((1,), jnp.int32), jnp.cumsum(out_lengths).astype(jnp.int32)]
    )
    return out_values, out_lengths, out_offsets


# speedup vs baseline: 30.8767x; 1.0022x over previous
import functools

import jax
import jax.numpy as jnp
from jax import lax
from jax.experimental import pallas as pl
from jax.experimental.pallas import tpu as pltpu
from jax.experimental.pallas import tpu_sc as plsc

_B = 16
_L = 2048
_D = 256
_SEQ = 2 * _L + 1
_ROWS = _B * _SEQ
_HALF = _L // 2
_C = 32
_K = _HALF // _C
_NBUF = 6
_PIPE = 4


def _pipelined_chunks(chunk_specs):
    K = len(chunk_specs)
    gathers = {}
    scatters = {}

    def start_gathers(k):
        gathers[k] = [g(k % _NBUF) for g in chunk_specs[k][0]]

    def start_scatters(k):
        scatters[k] = [s(k % _NBUF) for s in chunk_specs[k][1]]

    for k in range(min(_PIPE, K)):
        start_gathers(k)
    waited = set()
    for k in range(K):
        for cp in gathers[k]:
            cp.wait()
        start_scatters(k)
        nk = k + _PIPE
        if nk < K:
            if nk >= _NBUF:
                for cp in scatters[nk - _NBUF]:
                    cp.wait()
                waited.add(nk - _NBUF)
            start_gathers(nk)
    for k in range(K):
        if k not in waited:
            for cp in scatters[k]:
                cp.wait()


def _iota16():
    return lax.iota(jnp.int32, 16)


def _sc_body(item_hbm, action_hbm, ctx_hbm, out_hbm, *refs):
    bufs_i = list(refs[0:_NBUF])
    bufs_a = list(refs[_NBUF:2 * _NBUF])
    idx_i = list(refs[2 * _NBUF:3 * _NBUF])
    idx_a = list(refs[3 * _NBUF:4 * _NBUF])
    buf_c = refs[4 * _NBUF]
    idx_c = refs[4 * _NBUF + 1]
    gsems = list(refs[4 * _NBUF + 2:4 * _NBUF + 2 + _NBUF])
    ssems = list(refs[4 * _NBUF + 2 + _NBUF:4 * _NBUF + 2 + 2 * _NBUF])
    csem = refs[4 * _NBUF + 2 + 2 * _NBUF]

    c = lax.axis_index("c")
    s = lax.axis_index("s")
    w = s * 2 + c
    b = w // 2
    h = w % 2
    s0 = b * _L + h * _HALF
    dbase = b * _SEQ + 1 + h * _L

    @pl.when(w == 0)
    def _():
        cp = pltpu.make_async_copy(ctx_hbm, buf_c, csem)
        cp.start()
        idx_c[pl.ds(0, 16)] = _SEQ * _iota16()
        cp.wait()
        cp2 = pltpu.make_async_copy(buf_c, out_hbm.at[idx_c], csem)
        cp2.start()
        cp2.wait()

    def gather(src, base, bufs, slot):
        cp = pltpu.make_async_copy(
            src.at[pl.ds(pl.multiple_of(base, 8), _C)], bufs[slot], gsems[slot])
        cp.start()
        return cp

    def scatter(bufs, idxs, dst0, step_off, slot):
        for v in range(_C // 16):
            idxs[slot][pl.ds(16 * v, 16)] = (
                dst0 + 2 * (16 * v + _iota16()) + step_off)
        cp = pltpu.make_async_copy(bufs[slot], out_hbm.at[idxs[slot]], ssems[slot])
        cp.start()
        return cp

    specs = []
    for k in range(_K):
        src0 = s0 + k * _C
        dst0 = dbase + 2 * k * _C
        specs.append((
            [
                functools.partial(gather, item_hbm, src0, bufs_i),
                functools.partial(gather, action_hbm, src0, bufs_a),
            ],
            [
                functools.partial(scatter, bufs_i, idx_i, dst0, 0),
                functools.partial(scatter, bufs_a, idx_a, dst0, 1),
            ],
        ))
    _pipelined_chunks(specs)


def kernel(item_values, action_values, contextual_values):
    mesh = plsc.VectorSubcoreMesh(core_axis_name="c", subcore_axis_name="s")
    run = functools.partial(
        pl.kernel,
        out_type=jax.ShapeDtypeStruct((_ROWS, _D), jnp.float32),
        mesh=mesh,
        scratch_types=(
            [pltpu.VMEM((_C, _D), jnp.float32)] * (2 * _NBUF)
            + [pltpu.VMEM((_C,), jnp.int32)] * (2 * _NBUF)
            + [pltpu.VMEM((_B, _D), jnp.float32)]
            + [pltpu.VMEM((_B,), jnp.int32)]
            + [pltpu.SemaphoreType.DMA] * (2 * _NBUF + 1)
        ),
    )(_sc_body)
    out_values = run(item_values, action_values, contextual_values)
    out_lengths = jnp.full((_B,), _SEQ, dtype=jnp.int32)
    out_offsets = jnp.concatenate(
        [jnp.zeros((1,), jnp.int32), jnp.cumsum(out_lengths).astype(jnp.int32)]
    )
    return out_values, out_lengths, out_offsets


# speedup vs baseline: 30.9707x; 1.0030x over previous
import functools

import jax
import jax.numpy as jnp
from jax import lax
from jax.experimental import pallas as pl
from jax.experimental.pallas import tpu as pltpu
from jax.experimental.pallas import tpu_sc as plsc

_B = 16
_L = 2048
_D = 256
_SEQ = 2 * _L + 1
_ROWS = _B * _SEQ
_HALF = _L // 2
_C = 32
_K = _HALF // _C
_NBUF = 7
_PIPE = 5


def _pipelined_chunks(chunk_specs):
    K = len(chunk_specs)
    gathers = {}
    scatters = {}

    def start_gathers(k):
        gathers[k] = [g(k % _NBUF) for g in chunk_specs[k][0]]

    def start_scatters(k):
        scatters[k] = [s(k % _NBUF) for s in chunk_specs[k][1]]

    for k in range(min(_PIPE, K)):
        start_gathers(k)
    waited = set()
    for k in range(K):
        for cp in gathers[k]:
            cp.wait()
        start_scatters(k)
        nk = k + _PIPE
        if nk < K:
            if nk >= _NBUF:
                for cp in scatters[nk - _NBUF]:
                    cp.wait()
                waited.add(nk - _NBUF)
            start_gathers(nk)
    for k in range(K):
        if k not in waited:
            for cp in scatters[k]:
                cp.wait()


def _iota16():
    return lax.iota(jnp.int32, 16)


def _sc_body(item_hbm, action_hbm, ctx_hbm, out_hbm, *refs):
    bufs_i = list(refs[0:_NBUF])
    bufs_a = list(refs[_NBUF:2 * _NBUF])
    idx_i = list(refs[2 * _NBUF:3 * _NBUF])
    idx_a = list(refs[3 * _NBUF:4 * _NBUF])
    buf_c = refs[4 * _NBUF]
    idx_c = refs[4 * _NBUF + 1]
    gsems = list(refs[4 * _NBUF + 2:4 * _NBUF + 2 + _NBUF])
    ssems = list(refs[4 * _NBUF + 2 + _NBUF:4 * _NBUF + 2 + 2 * _NBUF])
    csem = refs[4 * _NBUF + 2 + 2 * _NBUF]

    c = lax.axis_index("c")
    s = lax.axis_index("s")
    w = s * 2 + c
    b = w // 2
    h = w % 2
    s0 = b * _L + h * _HALF
    dbase = b * _SEQ + 1 + h * _L

    @pl.when(w == 0)
    def _():
        cp = pltpu.make_async_copy(ctx_hbm, buf_c, csem)
        cp.start()
        idx_c[pl.ds(0, 16)] = _SEQ * _iota16()
        cp.wait()
        cp2 = pltpu.make_async_copy(buf_c, out_hbm.at[idx_c], csem)
        cp2.start()
        cp2.wait()

    def gather(src, base, bufs, slot):
        cp = pltpu.make_async_copy(
            src.at[pl.ds(pl.multiple_of(base, 8), _C)], bufs[slot], gsems[slot])
        cp.start()
        return cp

    def scatter(bufs, idxs, dst0, step_off, slot):
        for v in range(_C // 16):
            idxs[slot][pl.ds(16 * v, 16)] = (
                dst0 + 2 * (16 * v + _iota16()) + step_off)
        cp = pltpu.make_async_copy(bufs[slot], out_hbm.at[idxs[slot]], ssems[slot])
        cp.start()
        return cp

    specs = []
    for k in range(_K):
        src0 = s0 + k * _C
        dst0 = dbase + 2 * k * _C
        specs.append((
            [
                functools.partial(gather, item_hbm, src0, bufs_i),
                functools.partial(gather, action_hbm, src0, bufs_a),
            ],
            [
                functools.partial(scatter, bufs_i, idx_i, dst0, 0),
                functools.partial(scatter, bufs_a, idx_a, dst0, 1),
            ],
        ))
    _pipelined_chunks(specs)


def kernel(item_values, action_values, contextual_values):
    mesh = plsc.VectorSubcoreMesh(core_axis_name="c", subcore_axis_name="s")
    run = functools.partial(
        pl.kernel,
        out_type=jax.ShapeDtypeStruct((_ROWS, _D), jnp.float32),
        mesh=mesh,
        scratch_types=(
            [pltpu.VMEM((_C, _D), jnp.float32)] * (2 * _NBUF)
            + [pltpu.VMEM((_C,), jnp.int32)] * (2 * _NBUF)
            + [pltpu.VMEM((_B, _D), jnp.float32)]
            + [pltpu.VMEM((_B,), jnp.int32)]
            + [pltpu.SemaphoreType.DMA] * (2 * _NBUF + 1)
        ),
    )(_sc_body)
    out_values = run(item_values, action_values, contextual_values)
    out_lengths = jnp.full((_B,), _SEQ, dtype=jnp.int32)
    out_offsets = jnp.concatenate(
        [jnp.zeros((1,), jnp.int32), jnp.cumsum(out_lengths).astype(jnp.int32)]
    )
    return out_values, out_lengths, out_offsets
